# baseline (speedup 1.0000x reference)
import functools

import jax
import jax.numpy as jnp
from jax import lax
from jax.experimental import pallas as pl
from jax.experimental.pallas import tpu as pltpu
from jax.experimental.pallas import tpu_sc as plsc

K = 8192
D = 64
B = 8192
BLK = 256
MU = 0.25
_DP = 128


def _vq_block_kernel(x_ref, cb_ref, idx_ref, probs_ref, loss_ref,
                     cn_ref, cb16_ref):
    i = pl.program_id(0)

    @pl.when(i == 0)
    def _():
        cb = cb_ref[...]
        ones = jnp.ones((1, D), jnp.float32)
        cn_ref[...] = lax.dot_general(
            ones, cb * cb, (((1,), (1,)), ((), ())),
            precision=lax.Precision.HIGHEST,
            preferred_element_type=jnp.float32)
        cb16_ref[...] = cb.astype(jnp.bfloat16)

    x = x_ref[...]
    x2 = (x + x).astype(jnp.bfloat16)
    mm = lax.dot_general(x2, cb16_ref[...], (((1,), (1,)), ((), ())),
                         preferred_element_type=jnp.float32)
    u = mm - cn_ref[...]
    mu = jnp.max(u, axis=-1, keepdims=True)
    e = jnp.exp(u - mu)
    r = 1.0 / jnp.sum(e, axis=-1, keepdims=True)
    probs = e * r
    probs_ref[...] = probs
    iota = lax.broadcasted_iota(jnp.int32, (BLK, K), 1).astype(jnp.float32)
    cand = jnp.where(probs == r, iota, jnp.float32(K))
    idx_ref[...] = jnp.min(cand, axis=-1).astype(jnp.int32)
    xn = jnp.sum(x * x, axis=-1)
    loss_ref[...] = ((1.0 + MU) / D) * (xn - mu[:, 0])


def _distances_softmax(x, codebook_weight):
    grid = (B // BLK,)
    idx, probs, loss = pl.pallas_call(
        _vq_block_kernel,
        grid=grid,
        in_specs=[
            pl.BlockSpec((BLK, D), lambda i: (i, 0)),
            pl.BlockSpec((K, D), lambda i: (0, 0)),
        ],
        out_specs=[
            pl.BlockSpec((BLK,), lambda i: (i,)),
            pl.BlockSpec((BLK, K), lambda i: (i, 0)),
            pl.BlockSpec((BLK,), lambda i: (i,)),
        ],
        out_shape=[
            jax.ShapeDtypeStruct((B,), jnp.int32),
            jax.ShapeDtypeStruct((B, K), jnp.float32),
            jax.ShapeDtypeStruct((B,), jnp.float32),
        ],
        scratch_shapes=[
            pltpu.VMEM((1, K), jnp.float32),
            pltpu.VMEM((K, D), jnp.bfloat16),
        ],
        compiler_params=pltpu.CompilerParams(
            dimension_semantics=("arbitrary",),
        ),
    )(x, codebook_weight)
    return idx, probs, loss


_SC_GATHER = None


def _get_sc_gather():
    global _SC_GATHER
    if _SC_GATHER is not None:
        return _SC_GATHER
    info = plsc.get_sparse_core_info()
    nw = info.num_cores * info.num_subcores
    b_per_w = B // nw

    @functools.partial(
        pl.kernel,
        mesh=plsc.VectorSubcoreMesh(core_axis_name="c", subcore_axis_name="s"),
        out_type=jax.ShapeDtypeStruct((B, _DP), jnp.float32),
        scratch_types=[
            pltpu.VMEM((b_per_w,), jnp.int32),
            pltpu.VMEM((b_per_w, _DP), jnp.float32),
            pltpu.SemaphoreType.DMA,
        ],
    )
    def _sc_gather(table_hbm, idx_hbm, out_hbm, idx_v, rows_v, sem):
        wid = lax.axis_index("s") * info.num_cores + lax.axis_index("c")
        base = wid * b_per_w
        pltpu.sync_copy(idx_hbm.at[pl.ds(base, b_per_w)], idx_v)
        pltpu.async_copy(table_hbm.at[idx_v], rows_v, sem).wait()
        pltpu.sync_copy(rows_v, out_hbm.at[pl.ds(base, b_per_w)])

    _SC_GATHER = _sc_gather
    return _SC_GATHER


def kernel(x, codebook_weight):
    idx, probs, loss = _distances_softmax(x, codebook_weight)
    table = jnp.pad(codebook_weight, ((0, 0), (0, _DP - D)))
    quant = _get_sc_gather()(table, idx)[:, :D]
    return (quant, idx, probs, loss)


# speedup vs baseline: 1.0294x; 1.0294x over previous
import functools

import jax
import jax.numpy as jnp
from jax import lax
from jax.experimental import pallas as pl
from jax.experimental.pallas import tpu as pltpu
from jax.experimental.pallas import tpu_sc as plsc

K = 8192
D = 64
B = 8192
BLK = 256
MU = 0.25
_DP = 128


def _vq_block_kernel(x_ref, cb_ref, idx_ref, probs_ref, loss_ref, tab_ref,
                     cn_ref):
    i = pl.program_id(0)

    @pl.when(i == 0)
    def _():
        cb = cb_ref[...]
        ones = jnp.ones((1, D), jnp.float32)
        cn_ref[...] = lax.dot_general(
            ones, cb * cb, (((1,), (1,)), ((), ())),
            precision=lax.Precision.HIGHEST,
            preferred_element_type=jnp.float32)
        tab_ref[:, 0:D] = cb
        tab_ref[:, D:_DP] = jnp.zeros((K, _DP - D), jnp.float32)

    x = x_ref[...]
    x2 = x + x
    mm = lax.dot_general(x2, cb_ref[...], (((1,), (1,)), ((), ())),
                         preferred_element_type=jnp.float32)
    u = mm - cn_ref[...]
    mu = jnp.max(u, axis=-1, keepdims=True)
    e = jnp.exp(u - mu)
    r = 1.0 / jnp.sum(e, axis=-1, keepdims=True)
    probs = e * r
    probs_ref[...] = probs
    iota = lax.broadcasted_iota(jnp.int32, (BLK, K), 1).astype(jnp.float32)
    cand = jnp.where(probs == r, iota, jnp.float32(K))
    idx_ref[...] = jnp.min(cand, axis=-1).astype(jnp.int32)
    xn = jnp.sum(x * x, axis=-1)
    loss_ref[...] = ((1.0 + MU) / D) * (xn - mu[:, 0])


def _distances_softmax(x, codebook_weight):
    grid = (B // BLK,)
    idx, probs, loss, tab = pl.pallas_call(
        _vq_block_kernel,
        grid=grid,
        in_specs=[
            pl.BlockSpec((BLK, D), lambda i: (i, 0)),
            pl.BlockSpec((K, D), lambda i: (0, 0)),
        ],
        out_specs=[
            pl.BlockSpec((BLK,), lambda i: (i,)),
            pl.BlockSpec((BLK, K), lambda i: (i, 0)),
            pl.BlockSpec((BLK,), lambda i: (i,)),
            pl.BlockSpec((K, _DP), lambda i: (0, 0)),
        ],
        out_shape=[
            jax.ShapeDtypeStruct((B,), jnp.int32),
            jax.ShapeDtypeStruct((B, K), jnp.float32),
            jax.ShapeDtypeStruct((B,), jnp.float32),
            jax.ShapeDtypeStruct((K, _DP), jnp.float32),
        ],
        scratch_shapes=[
            pltpu.VMEM((1, K), jnp.float32),
        ],
        compiler_params=pltpu.CompilerParams(
            dimension_semantics=("arbitrary",),
        ),
    )(x, codebook_weight)
    return idx, probs, loss, tab


_SC_GATHER = None


def _get_sc_gather():
    global _SC_GATHER
    if _SC_GATHER is not None:
        return _SC_GATHER
    info = plsc.get_sparse_core_info()
    nw = info.num_cores * info.num_subcores
    b_per_w = B // nw

    @functools.partial(
        pl.kernel,
        mesh=plsc.VectorSubcoreMesh(core_axis_name="c", subcore_axis_name="s"),
        out_type=jax.ShapeDtypeStruct((B, _DP), jnp.float32),
        scratch_types=[
            pltpu.VMEM((b_per_w,), jnp.int32),
            pltpu.VMEM((b_per_w, _DP), jnp.float32),
            pltpu.SemaphoreType.DMA,
        ],
    )
    def _sc_gather(table_hbm, idx_hbm, out_hbm, idx_v, rows_v, sem):
        wid = lax.axis_index("s") * info.num_cores + lax.axis_index("c")
        base = wid * b_per_w
        pltpu.sync_copy(idx_hbm.at[pl.ds(base, b_per_w)], idx_v)
        pltpu.async_copy(table_hbm.at[idx_v], rows_v, sem).wait()
        pltpu.sync_copy(rows_v, out_hbm.at[pl.ds(base, b_per_w)])

    _SC_GATHER = _sc_gather
    return _SC_GATHER


def kernel(x, codebook_weight):
    idx, probs, loss, table = _distances_softmax(x, codebook_weight)
    quant = _get_sc_gather()(table, idx)[:, :D]
    return (quant, idx, probs, loss)
